# baseline (speedup 1.0000x reference)
import functools

import jax
import jax.numpy as jnp
from jax import lax
from jax.experimental import pallas as pl
from jax.experimental.pallas import tpu as pltpu
from jax.experimental.pallas import tpu_sc as plsc

_NC = 2
_NS = 16
_NW = _NC * _NS
_LANES = 16
_K = 64
_EPS = 1e-5


def _sc_gather(fps0, fps1, fps2, fps3, F0p, F1p, F2, F3):
    B, S = fps3.shape
    N1 = fps0.shape[1]
    N0 = N1 * 2
    N2 = fps1.shape[1]
    N3 = fps2.shape[1]
    C = F2.shape[1]
    H = C // 2
    rows_pw = (B * S) // _NW
    halves = S // rows_pw
    nsub = rows_pw // _K

    mesh = plsc.VectorSubcoreMesh(
        core_axis_name="c", subcore_axis_name="s",
        num_cores=_NC, num_subcores=_NS)

    @functools.partial(
        pl.kernel, mesh=mesh,
        compiler_params=pltpu.CompilerParams(
            needs_layout_passes=False, use_tc_tiling_on_sc=True),
        out_type=tuple(
            jax.ShapeDtypeStruct((B * S, C), jnp.float32) for _ in range(4)),
        scratch_types=[
            pltpu.VMEM((N1,), jnp.int32),
            pltpu.VMEM((N2,), jnp.int32),
            pltpu.VMEM((N3,), jnp.int32),
            pltpu.VMEM((rows_pw,), jnp.int32),
            pltpu.VMEM((rows_pw,), jnp.int32),
            pltpu.VMEM((rows_pw,), jnp.int32),
            pltpu.VMEM((rows_pw,), jnp.int32),
            pltpu.VMEM((rows_pw,), jnp.int32),
            pltpu.VMEM((rows_pw,), jnp.int32),
            pltpu.VMEM((rows_pw,), jnp.int32),
            [[pltpu.VMEM((_K, C), jnp.float32) for _ in range(4)]
             for _ in range(2)],
            [pltpu.SemaphoreType.DMA for _ in range(2)],
            [pltpu.SemaphoreType.DMA for _ in range(2)],
        ],
    )
    def k(fps0_h, fps1_h, fps2_h, fps3_h, F0_h, F1_h, F2_h, F3_h,
          G0_h, G1_h, G2_h, G3_h,
          t0, t1, t2, i3, gi3, gi2, gi1, gi0, ho0, ho1, bufs, sg, sw):
        wid = lax.axis_index("s") * _NC + lax.axis_index("c")
        b = wid // halves
        base_local = (wid % halves) * rows_pw
        out_base = b * S + base_local

        pltpu.sync_copy(fps0_h.at[b], t0)
        pltpu.sync_copy(fps1_h.at[b], t1)
        pltpu.sync_copy(fps2_h.at[b], t2)
        pltpu.sync_copy(fps3_h.at[b, pl.ds(base_local, rows_pw)], i3)

        def chain(j, carry):
            off = j * _LANES
            v3 = i3[pl.ds(off, _LANES)]
            v2 = plsc.load_gather(t2, [v3])
            v1 = plsc.load_gather(t1, [v2])
            v0 = plsc.load_gather(t0, [v1])
            gi3[pl.ds(off, _LANES)] = v3 + b * N3
            gi2[pl.ds(off, _LANES)] = v2 + b * N2
            gi1[pl.ds(off, _LANES)] = lax.shift_right_logical(v1, 1) + b * (N1 // 2)
            gi0[pl.ds(off, _LANES)] = lax.shift_right_logical(v0, 1) + b * (N0 // 2)
            ho1[pl.ds(off, _LANES)] = ((v1 & 1) ^ 1) * H
            ho0[pl.ds(off, _LANES)] = ((v0 & 1) ^ 1) * H
            return carry
        lax.fori_loop(0, rows_pw // _LANES, chain, 0, unroll=4)

        tabs = (F0_h, F1_h, F2_h, F3_h)
        outs = (G0_h, G1_h, G2_h, G3_h)
        gis = (gi0, gi1, gi2, gi3)
        hos = (ho0, ho1)
        iota16 = lax.iota(jnp.int32, 16)
        zeros16 = jnp.zeros((_LANES,), jnp.float32)

        def fire_gathers(sub, v):
            off = sub * _K
            for t in range(4):
                pltpu.async_copy(
                    tabs[t].at[gis[t].at[pl.ds(off, _K)]], bufs[v][t], sg[v])

        def wait_gathers(v):
            for t in range(4):
                pltpu.make_async_copy(
                    tabs[t].at[pl.ds(0, _K)], bufs[v][t], sg[v]).wait()

        def zero_dead_halves(sub, v):
            off = sub * _K
            for t in range(2):
                buf = bufs[v][t]
                ho = hos[t]
                def zrow(g, carry):
                    rows = g * _LANES + iota16
                    hh = ho[pl.ds(off + g * _LANES, _LANES)]
                    def zcol(cc, carry2):
                        plsc.store_scatter(buf, [rows, hh + cc], zeros16)
                        return carry2
                    lax.fori_loop(0, H, zcol, 0, unroll=16)
                    return carry
                lax.fori_loop(0, _K // _LANES, zrow, 0)

        def fire_writes(sub, v):
            off = out_base + sub * _K
            for t in range(4):
                pltpu.async_copy(bufs[v][t], outs[t].at[pl.ds(off, _K)],
                                 sw[v])

        def wait_writes(v):
            for t in range(4):
                pltpu.make_async_copy(
                    bufs[v][t], outs[t].at[pl.ds(0, _K)], sw[v]).wait()

        fire_gathers(0, 0)

        def gstep(i2, carry):
            v = lax.rem(i2, 2)
            @pl.when(v == 0)
            def _():
                wait_gathers(0)
                zero_dead_halves(i2, 0)
                fire_writes(i2, 0)
            @pl.when(v == 1)
            def _():
                wait_gathers(1)
                zero_dead_halves(i2, 1)
                fire_writes(i2, 1)
            @pl.when(i2 >= 1)
            def _():
                @pl.when(v == 0)
                def _():
                    wait_writes(1)
                @pl.when(v == 1)
                def _():
                    wait_writes(0)
            @pl.when(i2 + 1 < nsub)
            def _():
                @pl.when(v == 0)
                def _():
                    fire_gathers(i2 + 1, 1)
                @pl.when(v == 1)
                def _():
                    fire_gathers(i2 + 1, 0)
            return carry
        lax.fori_loop(0, nsub, gstep, 0)
        if nsub % 2 == 0:
            wait_writes(1)
        else:
            wait_writes(0)

    return k(fps0, fps1, fps2, fps3, F0p, F1p, F2, F3)


_CHUNK = 2048


def _leaky(z):
    return jnp.where(z >= 0, z, 0.2 * z)


def _fused_body(nrows, pred, g0, g1, g2, g3, f4c, w0, w1, w2, w3, w4,
                gam4, bet4, g4v, be4v, out,
                s_scr, sums, sumsq, s4s, s4q):
    p = pl.program_id(0)
    i = pl.program_id(1)
    inv_n = 1.0 / nrows

    @pl.when(jnp.logical_and(p == 0, i == 0))
    def _init():
        sums[...] = jnp.zeros_like(sums)
        sumsq[...] = jnp.zeros_like(sumsq)
        s4s[...] = jnp.zeros_like(s4s)
        s4q[...] = jnp.zeros_like(s4q)

    pairs = ((g0, w0), (g1, w1), (g2, w2), (g3, w3))

    @pl.when(p == 0)
    def _stats():
        for k, (g, w) in enumerate(pairs):
            o = jnp.dot(g[...], w[...], preferred_element_type=jnp.float32)
            sums[k:k + 1, :] += jnp.sum(o, axis=0, keepdims=True)
            sumsq[k:k + 1, :] += jnp.sum(o * o, axis=0, keepdims=True)

    @pl.when(p == 1)
    def _apply():
        acc = f4c[...]
        for k, (g, w) in enumerate(pairs):
            o = jnp.dot(g[...], w[...], preferred_element_type=jnp.float32)
            m = sums[k:k + 1, :] * inv_n
            var = sumsq[k:k + 1, :] * inv_n - m * m
            a = gam4[k:k + 1, :] * lax.rsqrt(var + _EPS)
            acc += _leaky(a * (o - m) + bet4[k:k + 1, :])
        s_scr[pl.ds(i * _CHUNK, _CHUNK), :] = acc
        o4 = jnp.dot(acc, w4[...], preferred_element_type=jnp.float32)
        s4s[...] += jnp.sum(o4, axis=0, keepdims=True)
        s4q[...] += jnp.sum(o4 * o4, axis=0, keepdims=True)

    @pl.when(p == 2)
    def _final():
        sc = s_scr[pl.ds(i * _CHUNK, _CHUNK), :]
        o = jnp.dot(sc, w4[...], preferred_element_type=jnp.float32)
        m = s4s[...] * inv_n
        var = s4q[...] * inv_n - m * m
        a = g4v[...] * lax.rsqrt(var + _EPS)
        res = _leaky(a * (o - m) + be4v[...]) + f4c[...]
        out[...] = jnp.where(pred[0, 0] != 0, res, f4c[...])


def _tc_fused(G0, G1, G2, G3, f4r, W0T, W1T, W2T, W3T, W4T,
              gam4, bet4, g4v, be4v, pred):
    R, C4 = f4r.shape
    nchunks = R // _CHUNK
    blk = lambda c: pl.BlockSpec(
        (_CHUNK, c), lambda p, i: (jnp.where(p == 2, 0, i), 0))
    f4blk = pl.BlockSpec(
        (_CHUNK, C4), lambda p, i: (jnp.where(p == 0, 0, i), 0))
    cblk = lambda a: pl.BlockSpec(a.shape, lambda p, i: (0, 0))
    return pl.pallas_call(
        functools.partial(_fused_body, float(R)),
        grid=(3, nchunks),
        in_specs=[pl.BlockSpec(memory_space=pltpu.SMEM),
                  blk(G0.shape[1]), blk(G1.shape[1]), blk(G2.shape[1]),
                  blk(G3.shape[1]), f4blk,
                  cblk(W0T), cblk(W1T), cblk(W2T), cblk(W3T), cblk(W4T),
                  cblk(gam4), cblk(bet4), cblk(g4v), cblk(be4v)],
        out_specs=pl.BlockSpec(
            (_CHUNK, C4), lambda p, i: (jnp.where(p == 2, i, 0), 0)),
        out_shape=jax.ShapeDtypeStruct((R, C4), jnp.float32),
        scratch_shapes=[
            pltpu.VMEM((R, C4), jnp.float32),
            pltpu.VMEM((4, C4), jnp.float32),
            pltpu.VMEM((4, C4), jnp.float32),
            pltpu.VMEM((1, C4), jnp.float32),
            pltpu.VMEM((1, C4), jnp.float32),
        ],
        compiler_params=pltpu.CompilerParams(
            dimension_semantics=("arbitrary", "arbitrary"),
            vmem_limit_bytes=112 * 1024 * 1024),
    )(pred, G0, G1, G2, G3, f4r, W0T, W1T, W2T, W3T, W4T,
      gam4, bet4, g4v, be4v)


def kernel(num_point, f0, f1, f2, f3, f4, FPS_0, FPS_1, FPS_2, FPS_3,
           W04, b04, g04, be04, W14, b14, g14, be14, W24, b24, g24, be24,
           W34, b34, g34, be34, W4, b4, g4, be4):
    B, N0, C0 = f0.shape
    S = FPS_3.shape[1]
    C4 = f4.shape[2]

    F0p = f0.reshape(B * (N0 // 2), 2 * C0)
    F1p = f1.reshape(B * (f1.shape[1] // 2), 2 * f1.shape[2])
    F2 = f2.reshape(B * f2.shape[1], f2.shape[2])
    F3 = f3.reshape(B * f3.shape[1], f3.shape[2])

    G0, G1, G2, G3 = _sc_gather(
        FPS_0.astype(jnp.int32), FPS_1.astype(jnp.int32),
        FPS_2.astype(jnp.int32), FPS_3.astype(jnp.int32),
        F0p, F1p, F2, F3)

    W0T = jnp.concatenate([W04.T, W04.T], axis=0)
    W1T = jnp.concatenate([W14.T, W14.T], axis=0)
    W2T, W3T, W4T = W24.T, W34.T, W4.T
    gam4 = jnp.stack([g04, g14, g24, g34])
    bet4 = jnp.stack([be04, be14, be24, be34])
    f4r = f4.reshape(B * S, C4)

    pred = (jnp.asarray(num_point, jnp.int32) == 128).astype(jnp.int32)
    f4new = _tc_fused(G0, G1, G2, G3, f4r, W0T, W1T, W2T, W3T, W4T,
                      gam4, bet4, g4.reshape(1, C4), be4.reshape(1, C4),
                      pred.reshape(1, 1))
    return (f0, f1, f2, f3, f4new.reshape(B, S, C4))


# speedup vs baseline: 1.2337x; 1.2337x over previous
import functools

import jax
import jax.numpy as jnp
from jax import lax
from jax.experimental import pallas as pl
from jax.experimental.pallas import tpu as pltpu
from jax.experimental.pallas import tpu_sc as plsc

_NC = 2
_NS = 16
_NW = _NC * _NS
_LANES = 16
_K = 64
_EPS = 1e-5


def _pack_body(x1, x2, o):
    o[...] = jnp.concatenate([x1[...].T, x2[...].T], axis=1)


def _tc_pack(ft, B, C, N, nck=2048):
    Nh = N // 2
    nblk = Nh // nck
    return pl.pallas_call(
        _pack_body,
        grid=(B, nblk),
        in_specs=[
            pl.BlockSpec((C, nck), lambda b, i: (b, i)),
            pl.BlockSpec((C, nck), lambda b, i: (b, i + nblk)),
        ],
        out_specs=pl.BlockSpec((nck, 2 * C), lambda b, i: (b * nblk + i, 0)),
        out_shape=jax.ShapeDtypeStruct((B * Nh, 2 * C), jnp.float32),
    )(ft, ft)


def _sc_gather(fps0, fps1, fps2, fps3, F0p, F1p, F2, F3):
    B, S = fps3.shape
    N1 = fps0.shape[1]
    N0 = N1 * 2
    N2 = fps1.shape[1]
    N3 = fps2.shape[1]
    C = F2.shape[1]
    H = C // 2
    rows_pw = (B * S) // _NW
    halves = S // rows_pw
    nsub = rows_pw // _K

    mesh = plsc.VectorSubcoreMesh(
        core_axis_name="c", subcore_axis_name="s",
        num_cores=_NC, num_subcores=_NS)

    @functools.partial(
        pl.kernel, mesh=mesh,
        compiler_params=pltpu.CompilerParams(
            needs_layout_passes=False, use_tc_tiling_on_sc=True),
        out_type=tuple(
            jax.ShapeDtypeStruct((B * S, C), jnp.float32) for _ in range(4)),
        scratch_types=[
            pltpu.VMEM((N1,), jnp.int32),
            pltpu.VMEM((N2,), jnp.int32),
            pltpu.VMEM((N3,), jnp.int32),
            pltpu.VMEM((rows_pw,), jnp.int32),
            pltpu.VMEM((rows_pw,), jnp.int32),
            pltpu.VMEM((rows_pw,), jnp.int32),
            pltpu.VMEM((rows_pw,), jnp.int32),
            pltpu.VMEM((rows_pw,), jnp.int32),
            pltpu.VMEM((rows_pw,), jnp.int32),
            pltpu.VMEM((rows_pw,), jnp.int32),
            [[pltpu.VMEM((_K, C), jnp.float32) for _ in range(4)]
             for _ in range(2)],
            [pltpu.SemaphoreType.DMA for _ in range(2)],
            [pltpu.SemaphoreType.DMA for _ in range(2)],
        ],
    )
    def k(fps0_h, fps1_h, fps2_h, fps3_h, F0_h, F1_h, F2_h, F3_h,
          G0_h, G1_h, G2_h, G3_h,
          t0, t1, t2, i3, gi3, gi2, gi1, gi0, ho0, ho1, bufs, sg, sw):
        wid = lax.axis_index("s") * _NC + lax.axis_index("c")
        b = wid // halves
        base_local = (wid % halves) * rows_pw
        out_base = b * S + base_local

        pltpu.sync_copy(fps0_h.at[b], t0)
        pltpu.sync_copy(fps1_h.at[b], t1)
        pltpu.sync_copy(fps2_h.at[b], t2)
        pltpu.sync_copy(fps3_h.at[b, pl.ds(base_local, rows_pw)], i3)

        def chain(j, carry):
            off = j * _LANES
            v3 = i3[pl.ds(off, _LANES)]
            v2 = plsc.load_gather(t2, [v3])
            v1 = plsc.load_gather(t1, [v2])
            v0 = plsc.load_gather(t0, [v1])
            gi3[pl.ds(off, _LANES)] = v3 + b * N3
            gi2[pl.ds(off, _LANES)] = v2 + b * N2
            gi1[pl.ds(off, _LANES)] = (v1 & (N1 // 2 - 1)) + b * (N1 // 2)
            gi0[pl.ds(off, _LANES)] = (v0 & (N0 // 2 - 1)) + b * (N0 // 2)
            sh1 = (N1 // 2).bit_length() - 1
            sh0 = (N0 // 2).bit_length() - 1
            ho1[pl.ds(off, _LANES)] = (lax.shift_right_logical(v1, sh1) ^ 1) * H
            ho0[pl.ds(off, _LANES)] = (lax.shift_right_logical(v0, sh0) ^ 1) * H
            return carry
        lax.fori_loop(0, rows_pw // _LANES, chain, 0, unroll=4)

        tabs = (F0_h, F1_h, F2_h, F3_h)
        outs = (G0_h, G1_h, G2_h, G3_h)
        gis = (gi0, gi1, gi2, gi3)
        hos = (ho0, ho1)
        iota16 = lax.iota(jnp.int32, 16)
        zeros16 = jnp.zeros((_LANES,), jnp.float32)

        def fire_gathers(sub, v):
            off = sub * _K
            for t in range(4):
                pltpu.async_copy(
                    tabs[t].at[gis[t].at[pl.ds(off, _K)]], bufs[v][t], sg[v])

        def wait_gathers(v):
            for t in range(4):
                pltpu.make_async_copy(
                    tabs[t].at[pl.ds(0, _K)], bufs[v][t], sg[v]).wait()

        def zero_dead_halves(sub, v):
            off = sub * _K
            for t in range(2):
                buf = bufs[v][t]
                ho = hos[t]
                def zrow(g, carry):
                    rows = g * _LANES + iota16
                    hh = ho[pl.ds(off + g * _LANES, _LANES)]
                    def zcol(cc, carry2):
                        plsc.store_scatter(buf, [rows, hh + cc], zeros16)
                        return carry2
                    lax.fori_loop(0, H, zcol, 0, unroll=16)
                    return carry
                lax.fori_loop(0, _K // _LANES, zrow, 0)

        def fire_writes(sub, v):
            off = out_base + sub * _K
            for t in range(4):
                pltpu.async_copy(bufs[v][t], outs[t].at[pl.ds(off, _K)],
                                 sw[v])

        def wait_writes(v):
            for t in range(4):
                pltpu.make_async_copy(
                    bufs[v][t], outs[t].at[pl.ds(0, _K)], sw[v]).wait()

        fire_gathers(0, 0)

        def gstep(i2, carry):
            v = lax.rem(i2, 2)
            @pl.when(v == 0)
            def _():
                wait_gathers(0)
                zero_dead_halves(i2, 0)
                fire_writes(i2, 0)
            @pl.when(v == 1)
            def _():
                wait_gathers(1)
                zero_dead_halves(i2, 1)
                fire_writes(i2, 1)
            @pl.when(i2 >= 1)
            def _():
                @pl.when(v == 0)
                def _():
                    wait_writes(1)
                @pl.when(v == 1)
                def _():
                    wait_writes(0)
            @pl.when(i2 + 1 < nsub)
            def _():
                @pl.when(v == 0)
                def _():
                    fire_gathers(i2 + 1, 1)
                @pl.when(v == 1)
                def _():
                    fire_gathers(i2 + 1, 0)
            return carry
        lax.fori_loop(0, nsub, gstep, 0)
        if nsub % 2 == 0:
            wait_writes(1)
        else:
            wait_writes(0)

    return k(fps0, fps1, fps2, fps3, F0p, F1p, F2, F3)


_CHUNK = 2048


def _leaky(z):
    return jnp.where(z >= 0, z, 0.2 * z)


def _fused_body(nrows, pred, g0, g1, g2, g3, f4c, w0, w1, w2, w3, w4,
                gam4, bet4, g4v, be4v, out,
                s_scr, sums, sumsq, s4s, s4q):
    p = pl.program_id(0)
    i = pl.program_id(1)
    inv_n = 1.0 / nrows

    @pl.when(jnp.logical_and(p == 0, i == 0))
    def _init():
        sums[...] = jnp.zeros_like(sums)
        sumsq[...] = jnp.zeros_like(sumsq)
        s4s[...] = jnp.zeros_like(s4s)
        s4q[...] = jnp.zeros_like(s4q)

    pairs = ((g0, w0), (g1, w1), (g2, w2), (g3, w3))

    @pl.when(p == 0)
    def _stats():
        for k, (g, w) in enumerate(pairs):
            o = jnp.dot(g[...], w[...], preferred_element_type=jnp.float32)
            sums[k:k + 1, :] += jnp.sum(o, axis=0, keepdims=True)
            sumsq[k:k + 1, :] += jnp.sum(o * o, axis=0, keepdims=True)

    @pl.when(p == 1)
    def _apply():
        acc = f4c[...]
        for k, (g, w) in enumerate(pairs):
            o = jnp.dot(g[...], w[...], preferred_element_type=jnp.float32)
            m = sums[k:k + 1, :] * inv_n
            var = sumsq[k:k + 1, :] * inv_n - m * m
            a = gam4[k:k + 1, :] * lax.rsqrt(var + _EPS)
            acc += _leaky(a * (o - m) + bet4[k:k + 1, :])
        s_scr[pl.ds(i * _CHUNK, _CHUNK), :] = acc
        o4 = jnp.dot(acc, w4[...], preferred_element_type=jnp.float32)
        s4s[...] += jnp.sum(o4, axis=0, keepdims=True)
        s4q[...] += jnp.sum(o4 * o4, axis=0, keepdims=True)

    @pl.when(p == 2)
    def _final():
        sc = s_scr[pl.ds(i * _CHUNK, _CHUNK), :]
        o = jnp.dot(sc, w4[...], preferred_element_type=jnp.float32)
        m = s4s[...] * inv_n
        var = s4q[...] * inv_n - m * m
        a = g4v[...] * lax.rsqrt(var + _EPS)
        res = _leaky(a * (o - m) + be4v[...]) + f4c[...]
        out[...] = jnp.where(pred[0, 0] != 0, res, f4c[...])


def _tc_fused(G0, G1, G2, G3, f4r, W0T, W1T, W2T, W3T, W4T,
              gam4, bet4, g4v, be4v, pred):
    R, C4 = f4r.shape
    nchunks = R // _CHUNK
    blk = lambda c: pl.BlockSpec(
        (_CHUNK, c), lambda p, i: (jnp.where(p == 2, 0, i), 0))
    f4blk = pl.BlockSpec(
        (_CHUNK, C4), lambda p, i: (jnp.where(p == 0, 0, i), 0))
    cblk = lambda a: pl.BlockSpec(a.shape, lambda p, i: (0, 0))
    return pl.pallas_call(
        functools.partial(_fused_body, float(R)),
        grid=(3, nchunks),
        in_specs=[pl.BlockSpec(memory_space=pltpu.SMEM),
                  blk(G0.shape[1]), blk(G1.shape[1]), blk(G2.shape[1]),
                  blk(G3.shape[1]), f4blk,
                  cblk(W0T), cblk(W1T), cblk(W2T), cblk(W3T), cblk(W4T),
                  cblk(gam4), cblk(bet4), cblk(g4v), cblk(be4v)],
        out_specs=pl.BlockSpec(
            (_CHUNK, C4), lambda p, i: (jnp.where(p == 2, i, 0), 0)),
        out_shape=jax.ShapeDtypeStruct((R, C4), jnp.float32),
        scratch_shapes=[
            pltpu.VMEM((R, C4), jnp.float32),
            pltpu.VMEM((4, C4), jnp.float32),
            pltpu.VMEM((4, C4), jnp.float32),
            pltpu.VMEM((1, C4), jnp.float32),
            pltpu.VMEM((1, C4), jnp.float32),
        ],
        compiler_params=pltpu.CompilerParams(
            dimension_semantics=("arbitrary", "arbitrary"),
            vmem_limit_bytes=112 * 1024 * 1024),
    )(pred, G0, G1, G2, G3, f4r, W0T, W1T, W2T, W3T, W4T,
      gam4, bet4, g4v, be4v)


def kernel(num_point, f0, f1, f2, f3, f4, FPS_0, FPS_1, FPS_2, FPS_3,
           W04, b04, g04, be04, W14, b14, g14, be14, W24, b24, g24, be24,
           W34, b34, g34, be34, W4, b4, g4, be4):
    B, N0, C0 = f0.shape
    S = FPS_3.shape[1]
    C4 = f4.shape[2]

    ft0 = jnp.swapaxes(f0, 1, 2).reshape(B * C0, N0)
    N1f, C1 = f1.shape[1], f1.shape[2]
    ft1 = jnp.swapaxes(f1, 1, 2).reshape(B * C1, N1f)
    F0p = _tc_pack(ft0, B, C0, N0)
    F1p = _tc_pack(ft1, B, C1, N1f)
    F2 = f2.reshape(B * f2.shape[1], f2.shape[2])
    F3 = f3.reshape(B * f3.shape[1], f3.shape[2])

    G0, G1, G2, G3 = _sc_gather(
        FPS_0.astype(jnp.int32), FPS_1.astype(jnp.int32),
        FPS_2.astype(jnp.int32), FPS_3.astype(jnp.int32),
        F0p, F1p, F2, F3)

    W0T = jnp.concatenate([W04.T, W04.T], axis=0)
    W1T = jnp.concatenate([W14.T, W14.T], axis=0)
    W2T, W3T, W4T = W24.T, W34.T, W4.T
    gam4 = jnp.stack([g04, g14, g24, g34])
    bet4 = jnp.stack([be04, be14, be24, be34])
    f4r = f4.reshape(B * S, C4)

    pred = (jnp.asarray(num_point, jnp.int32) == 128).astype(jnp.int32)
    f4new = _tc_fused(G0, G1, G2, G3, f4r, W0T, W1T, W2T, W3T, W4T,
                      gam4, bet4, g4.reshape(1, C4), be4.reshape(1, C4),
                      pred.reshape(1, 1))
    return (f0, f1, f2, f3, f4new.reshape(B, S, C4))
